# baseline (speedup 1.0000x reference)
import jax
import jax.numpy as jnp
from jax import lax
from jax.experimental import pallas as pl
from jax.experimental.pallas import tpu as pltpu
from jax.experimental.pallas import tpu_sc as plsc


def _body(x, tgt, lens, c, sos, labels, x_out, tgt_out, lens_s, c_s):
    B, S, J = x.shape
    T_L = labels.shape[1]
    wid = lax.axis_index("s") * 2 + lax.axis_index("c")
    b = wid % B
    kind = wid // B

    pltpu.sync_copy(lens, lens_s)
    pltpu.sync_copy(c, c_s)
    lane = lax.iota(jnp.int32, 16)
    lb = jnp.max(jnp.where(lane == b, lens_s[...], 0), axis=0)
    cb = jnp.max(jnp.where(lane == b, c_s[...], 0), axis=0)

    @pl.when(kind == 0)
    def _():
        pltpu.sync_copy(x.at[b], x_out.at[b, pl.ds(1, S)])
        pltpu.sync_copy(sos.at[pl.ds(b, 1)], x_out.at[b, pl.ds(0, 1)])
        pltpu.sync_copy(sos.at[pl.ds(b, 1)], x_out.at[b, pl.ds(lb + 1, 1)])
        pltpu.sync_copy(labels.at[cb], x_out.at[b, pl.ds(lb + 2, T_L)])

    @pl.when(kind == 1)
    def _():
        pltpu.sync_copy(tgt.at[b], tgt_out.at[b])
        pltpu.sync_copy(sos.at[pl.ds(b, 1)], tgt_out.at[b, pl.ds(lb, 1)])
        pltpu.sync_copy(labels.at[cb], tgt_out.at[b, pl.ds(lb + 1, T_L)])
        pltpu.sync_copy(sos.at[pl.ds(b, 1)], tgt_out.at[b, pl.ds(lb + 1 + T_L, 1)])


def kernel(x, tgt, lens, c, sos, labels):
    B, S, J = x.shape
    run = pl.kernel(
        _body,
        out_type=(
            jax.ShapeDtypeStruct((B, S + 1, J), x.dtype),
            jax.ShapeDtypeStruct((B, S, J), tgt.dtype),
        ),
        mesh=plsc.VectorSubcoreMesh(core_axis_name="c", subcore_axis_name="s"),
        compiler_params=pltpu.CompilerParams(
            use_tc_tiling_on_sc=False, needs_layout_passes=False
        ),
        scratch_types=[
            pltpu.VMEM((B,), jnp.int32),
            pltpu.VMEM((B,), jnp.int32),
        ],
    )
    x_out, tgt_out = run(x, tgt, lens, c, sos, labels)
    return (x_out, tgt_out, labels)


# speedup vs baseline: 10.4890x; 10.4890x over previous
import jax
import jax.numpy as jnp
from jax import lax
from jax.experimental import pallas as pl
from jax.experimental.pallas import tpu as pltpu
from jax.experimental.pallas import tpu_sc as plsc

_CH = 64


def _body(x, tgt, lens, c, sos, labels, x_out, tgt_out,
          buf, win, lens_s, c_s, g0, g1, s0, s1, wsem):
    B, S, J = x.shape
    T_L = labels.shape[1]
    NCH = S // _CH
    gsem = (g0, g1)
    ssem = (s0, s1)

    wid = lax.axis_index("s") * 2 + lax.axis_index("c")
    b = wid % B
    kind = wid // B

    pltpu.sync_copy(lens, lens_s)
    pltpu.sync_copy(c, c_s)
    lane = lax.iota(jnp.int32, 16)
    lb = jnp.max(jnp.where(lane == b, lens_s[...], 0), axis=0)
    cb = jnp.max(jnp.where(lane == b, c_s[...], 0), axis=0)

    def run(src, dst, shift, with_tail_sos):
        wrows = T_L + 2 if with_tail_sos else T_L + 1
        wd = [
            pltpu.async_copy(sos.at[pl.ds(b, 1)], win.at[pl.ds(0, 1)], wsem),
            pltpu.async_copy(labels.at[cb], win.at[pl.ds(1, T_L)], wsem),
        ]
        if with_tail_sos:
            wd.append(pltpu.async_copy(sos.at[pl.ds(b, 1)],
                                       win.at[pl.ds(T_L + 1, 1)], wsem))
        else:
            wd.append(pltpu.async_copy(sos.at[pl.ds(b, 1)],
                                       dst.at[b, pl.ds(0, 1)], wsem))

        g = [None] * NCH
        s = [None] * NCH
        g[0] = pltpu.async_copy(src.at[b, pl.ds(0, _CH)], buf.at[0], gsem[0])
        for i in range(NCH):
            if i + 1 < NCH:
                if i >= 1:
                    s[i - 1].wait()
                g[i + 1] = pltpu.async_copy(
                    src.at[b, pl.ds((i + 1) * _CH, _CH)],
                    buf.at[(i + 1) % 2], gsem[(i + 1) % 2])
            g[i].wait()
            s[i] = pltpu.async_copy(
                buf.at[i % 2],
                dst.at[b, pl.ds(i * _CH + shift, _CH)], ssem[i % 2])
        s[NCH - 2].wait()
        s[NCH - 1].wait()
        for d in wd:
            d.wait()
        pltpu.sync_copy(win.at[pl.ds(0, wrows)],
                        dst.at[b, pl.ds(lb + shift, wrows)])

    @pl.when(kind == 0)
    def _():
        run(x, x_out, 1, False)

    @pl.when(kind == 1)
    def _():
        run(tgt, tgt_out, 0, True)


def kernel(x, tgt, lens, c, sos, labels):
    B, S, J = x.shape
    T_L = labels.shape[1]
    run = pl.kernel(
        _body,
        out_type=(
            jax.ShapeDtypeStruct((B, S + 1, J), x.dtype),
            jax.ShapeDtypeStruct((B, S, J), tgt.dtype),
        ),
        mesh=plsc.VectorSubcoreMesh(core_axis_name="c", subcore_axis_name="s"),
        compiler_params=pltpu.CompilerParams(
            use_tc_tiling_on_sc=False, needs_layout_passes=False
        ),
        scratch_types=[
            pltpu.VMEM((2, _CH, J), x.dtype),
            pltpu.VMEM((T_L + 2, J), x.dtype),
            pltpu.VMEM((B,), jnp.int32),
            pltpu.VMEM((B,), jnp.int32),
            pltpu.SemaphoreType.DMA,
            pltpu.SemaphoreType.DMA,
            pltpu.SemaphoreType.DMA,
            pltpu.SemaphoreType.DMA,
            pltpu.SemaphoreType.DMA,
        ],
    )
    x_out, tgt_out = run(x, tgt, lens, c, sos, labels)
    return (x_out, tgt_out, labels)
